# baseline (speedup 1.0000x reference)
import functools

import jax
import jax.numpy as jnp
from jax import lax
from jax.experimental import pallas as pl
from jax.experimental.pallas import tpu as pltpu
from jax.experimental.pallas import tpu_sc as plsc

VOCAB = 100000
EMB = 64
BLOCK = 2048
B = 64
T = 2048

NUM_CORES = 2
NUM_SUBCORES = 16
NW = NUM_CORES * NUM_SUBCORES
TPW = (B * T) // NW
CHUNK = 512
NCHUNKS = TPW // CHUNK


def _embed_body(x_hbm, tok_hbm, pos_hbm, out_hbm, idx_v, rows_v, pos_v, sem):
    wid = lax.axis_index("s") * NUM_CORES + lax.axis_index("c")
    base = wid * TPW
    pltpu.sync_copy(x_hbm.at[pl.ds(base, TPW)], idx_v)
    for c in range(NCHUNKS):
        p0 = (c * CHUNK) % T
        pltpu.async_copy(
            tok_hbm.at[idx_v.at[pl.ds(c * CHUNK, CHUNK)]], rows_v, sem
        ).wait()
        pltpu.sync_copy(pos_hbm.at[pl.ds(p0, CHUNK)], pos_v)

        def add_body(i, _):
            for j in range(EMB // 16):
                s = pl.ds(j * 16, 16)
                rows_v[i, s] = rows_v[i, s] + pos_v[i, s]
            return 0

        lax.fori_loop(0, CHUNK, add_body, 0)
        pltpu.sync_copy(rows_v, out_hbm.at[pl.ds(base + c * CHUNK, CHUNK)])


@jax.jit
def kernel(x, token_embedding_table, position_embedding_table):
    Bv, Tv = x.shape
    xf = x.reshape(Bv * Tv).astype(jnp.int32)
    pos = position_embedding_table[:Tv]
    mesh = plsc.VectorSubcoreMesh(core_axis_name="c", subcore_axis_name="s")
    out = pl.kernel(
        _embed_body,
        mesh=mesh,
        compiler_params=pltpu.CompilerParams(use_tc_tiling_on_sc=False),
        out_type=jax.ShapeDtypeStruct((Bv * Tv, EMB), jnp.float32),
        scratch_types=[
            pltpu.VMEM((TPW,), jnp.int32),
            pltpu.VMEM((CHUNK, EMB), jnp.float32),
            pltpu.VMEM((CHUNK, EMB), jnp.float32),
            pltpu.SemaphoreType.DMA,
        ],
    )(xf, token_embedding_table, pos)
    return out.reshape(Bv, Tv, EMB)


# speedup vs baseline: 1.1199x; 1.1199x over previous
import functools

import jax
import jax.numpy as jnp
from jax import lax
from jax.experimental import pallas as pl
from jax.experimental.pallas import tpu as pltpu
from jax.experimental.pallas import tpu_sc as plsc

VOCAB = 100000
EMB = 64
B = 64
T = 2048

NUM_CORES = 2
NUM_SUBCORES = 16
NW = NUM_CORES * NUM_SUBCORES
TPW = (B * T) // NW
CHUNK = 512
NPC = T // CHUNK
NK = 2 * NPC


def _embed_body(x_hbm, tok_hbm, pos_hbm, out_hbm,
                idx_v, rows0_v, rows1_v, pos_v, sem_g, sem_p, sem_s):
    wid = lax.axis_index("s") * NUM_CORES + lax.axis_index("c")
    base = wid * TPW
    rows = (rows0_v, rows1_v)

    pltpu.sync_copy(x_hbm.at[pl.ds(base, TPW)], idx_v)
    pltpu.async_copy(pos_hbm.at[pl.ds(0, CHUNK)], pos_v, sem_p)

    def tok_off(k):
        c, r = k // 2, k % 2
        return r * T + c * CHUNK

    def gather(k):
        off = tok_off(k)
        pltpu.async_copy(
            tok_hbm.at[idx_v.at[pl.ds(off, CHUNK)]], rows[k % 2], sem_g)

    gather(0)
    for k in range(NK):
        c, r = k // 2, k % 2
        if k + 1 < NK:
            if k >= 1:
                pltpu.make_async_copy(
                    rows[(k + 1) % 2],
                    out_hbm.at[pl.ds(base + tok_off(k - 1), CHUNK)],
                    sem_s).wait()
            gather(k + 1)
        if r == 0:
            pltpu.make_async_copy(
                pos_hbm.at[pl.ds(c * CHUNK, CHUNK)], pos_v, sem_p).wait()
        pltpu.make_async_copy(
            tok_hbm.at[idx_v.at[pl.ds(tok_off(k), CHUNK)]], rows[k % 2],
            sem_g).wait()

        rv = rows[k % 2]

        def add_body(i, _):
            for j in range(EMB // 16):
                s = pl.ds(j * 16, 16)
                rv[i, s] = rv[i, s] + pos_v[i, s]
            return 0

        lax.fori_loop(0, CHUNK, add_body, 0)

        if r == 1 and c + 1 < NPC:
            pltpu.async_copy(
                pos_hbm.at[pl.ds((c + 1) * CHUNK, CHUNK)], pos_v, sem_p)
        pltpu.async_copy(
            rows[k % 2], out_hbm.at[pl.ds(base + tok_off(k), CHUNK)], sem_s)
    pltpu.make_async_copy(
        rows[(NK - 2) % 2], out_hbm.at[pl.ds(base + tok_off(NK - 2), CHUNK)],
        sem_s).wait()
    pltpu.make_async_copy(
        rows[(NK - 1) % 2], out_hbm.at[pl.ds(base + tok_off(NK - 1), CHUNK)],
        sem_s).wait()


@jax.jit
def kernel(x, token_embedding_table, position_embedding_table):
    Bv, Tv = x.shape
    xf = x.reshape(Bv * Tv).astype(jnp.int32)
    pos = position_embedding_table[:Tv]
    mesh = plsc.VectorSubcoreMesh(core_axis_name="c", subcore_axis_name="s")
    out = pl.kernel(
        _embed_body,
        mesh=mesh,
        compiler_params=pltpu.CompilerParams(use_tc_tiling_on_sc=False),
        out_type=jax.ShapeDtypeStruct((Bv * Tv, EMB), jnp.float32),
        scratch_types=[
            pltpu.VMEM((TPW,), jnp.int32),
            pltpu.VMEM((CHUNK, EMB), jnp.float32),
            pltpu.VMEM((CHUNK, EMB), jnp.float32),
            pltpu.VMEM((CHUNK, EMB), jnp.float32),
            pltpu.SemaphoreType.DMA,
            pltpu.SemaphoreType.DMA,
            pltpu.SemaphoreType.DMA,
        ],
    )(xf, token_embedding_table, pos)
    return out.reshape(Bv, Tv, EMB)


# speedup vs baseline: 1.1260x; 1.0054x over previous
import functools

import jax
import jax.numpy as jnp
from jax import lax
from jax.experimental import pallas as pl
from jax.experimental.pallas import tpu as pltpu
from jax.experimental.pallas import tpu_sc as plsc

VOCAB = 100000
EMB = 64
B = 64
T = 2048

NUM_CORES = 2
NUM_SUBCORES = 16
NW = NUM_CORES * NUM_SUBCORES
ROWS_PW = B // NW
CHUNK = 512
NPC = T // CHUNK
NK = ROWS_PW * NPC


def _embed_body(x_hbm, tok_hbm, pos_hbm, out_hbm,
                idx_v, rows0_v, rows1_v, pos_v, sem_g, sem_p, sem_s):
    wid = lax.axis_index("s") * NUM_CORES + lax.axis_index("c")
    b0 = wid * ROWS_PW
    rows = (rows0_v, rows1_v)

    pltpu.sync_copy(x_hbm.at[pl.ds(b0, ROWS_PW)], idx_v)
    pltpu.async_copy(pos_hbm.at[pl.ds(0, CHUNK)], pos_v, sem_p)

    def item(k):
        return k // 2, k % 2

    def gather(k):
        c, r = item(k)
        pltpu.async_copy(
            tok_hbm.at[idx_v.at[r, pl.ds(c * CHUNK, CHUNK)]],
            rows[k % 2], sem_g)

    def store_desc(k, start):
        c, r = item(k)
        cp = pltpu.async_copy if start else pltpu.make_async_copy
        return cp(rows[k % 2],
                  out_hbm.at[b0 + r, pl.ds(c * CHUNK, CHUNK)], sem_s)

    gather(0)
    for k in range(NK):
        c, r = item(k)
        if k + 1 < NK:
            if k >= 1:
                store_desc(k - 1, start=False).wait()
            gather(k + 1)
        if r == 0:
            pltpu.make_async_copy(
                pos_hbm.at[pl.ds(c * CHUNK, CHUNK)], pos_v, sem_p).wait()
        c_, r_ = item(k)
        pltpu.make_async_copy(
            tok_hbm.at[idx_v.at[r_, pl.ds(c_ * CHUNK, CHUNK)]],
            rows[k % 2], sem_g).wait()

        rv = rows[k % 2]

        def add_body(i, _):
            for j in range(EMB // 16):
                s = pl.ds(j * 16, 16)
                rv[i, s] = rv[i, s] + pos_v[i, s]
            return 0

        lax.fori_loop(0, CHUNK, add_body, 0)

        if r == 1 and c + 1 < NPC:
            pltpu.async_copy(
                pos_hbm.at[pl.ds((c + 1) * CHUNK, CHUNK)], pos_v, sem_p)
        store_desc(k, start=True)
    store_desc(NK - 2, start=False).wait()
    store_desc(NK - 1, start=False).wait()


@jax.jit
def kernel(x, token_embedding_table, position_embedding_table):
    Bv, Tv = x.shape
    pos = position_embedding_table[:Tv]
    mesh = plsc.VectorSubcoreMesh(core_axis_name="c", subcore_axis_name="s")
    out = pl.kernel(
        _embed_body,
        mesh=mesh,
        compiler_params=pltpu.CompilerParams(use_tc_tiling_on_sc=False),
        out_type=jax.ShapeDtypeStruct((Bv, Tv, EMB), jnp.float32),
        scratch_types=[
            pltpu.VMEM((ROWS_PW, T), jnp.int32),
            pltpu.VMEM((CHUNK, EMB), jnp.float32),
            pltpu.VMEM((CHUNK, EMB), jnp.float32),
            pltpu.VMEM((CHUNK, EMB), jnp.float32),
            pltpu.SemaphoreType.DMA,
            pltpu.SemaphoreType.DMA,
            pltpu.SemaphoreType.DMA,
        ],
    )(x.astype(jnp.int32), token_embedding_table, pos)
    return out


# speedup vs baseline: 1.5884x; 1.4106x over previous
import functools

import jax
import jax.numpy as jnp
from jax import lax
from jax.experimental import pallas as pl
from jax.experimental.pallas import tpu as pltpu
from jax.experimental.pallas import tpu_sc as plsc

VOCAB = 100000
EMB = 64
B = 64
T = 2048

NUM_CORES = 2
NUM_SUBCORES = 16
NW = NUM_CORES * NUM_SUBCORES
FPW = EMB // NW


def _embed_body(tokT_hbm, x_hbm, posT_hbm, out_hbm,
                row_v, idx0_v, idx1_v, pos_v, o0_v, o1_v,
                sem_r, sem_i, sem_s):
    wid = lax.axis_index("s") * NUM_CORES + lax.axis_index("c")
    idx = (idx0_v, idx1_v)
    o = (o0_v, o1_v)

    for f in range(FPW):
        e = wid * FPW + f
        pltpu.async_copy(tokT_hbm.at[e, pl.ds(0, VOCAB)], row_v, sem_r)
        pltpu.sync_copy(posT_hbm.at[e, pl.ds(0, T)], pos_v)
        pltpu.async_copy(x_hbm.at[0, pl.ds(0, T)], idx0_v, sem_i)
        pltpu.make_async_copy(
            tokT_hbm.at[e, pl.ds(0, VOCAB)], row_v, sem_r).wait()

        for b in range(B):
            cur = b % 2
            if b + 1 < B:
                pltpu.async_copy(
                    x_hbm.at[b + 1, pl.ds(0, T)], idx[1 - cur], sem_i)
            pltpu.make_async_copy(
                x_hbm.at[b, pl.ds(0, T)], idx[cur], sem_i).wait()
            if b >= 2:
                pltpu.make_async_copy(
                    o[cur], out_hbm.at[b - 2, e, pl.ds(0, T)], sem_s).wait()

            iv = idx[cur]
            ov = o[cur]

            def body(i, _):
                s = pl.ds(i * 16, 16)
                g = plsc.load_gather(row_v, [iv[s]])
                ov[s] = g + pos_v[s]
                return 0

            lax.fori_loop(0, T // 16, body, 0)
            pltpu.async_copy(o[cur], out_hbm.at[b, e, pl.ds(0, T)], sem_s)
        pltpu.make_async_copy(
            o[(B - 2) % 2], out_hbm.at[B - 2, e, pl.ds(0, T)], sem_s).wait()
        pltpu.make_async_copy(
            o[(B - 1) % 2], out_hbm.at[B - 1, e, pl.ds(0, T)], sem_s).wait()


@jax.jit
def kernel(x, token_embedding_table, position_embedding_table):
    Bv, Tv = x.shape
    tokT = token_embedding_table.T
    posT = position_embedding_table[:Tv].T
    mesh = plsc.VectorSubcoreMesh(core_axis_name="c", subcore_axis_name="s")
    outT = pl.kernel(
        _embed_body,
        mesh=mesh,
        compiler_params=pltpu.CompilerParams(
            use_tc_tiling_on_sc=True, needs_layout_passes=False),
        out_type=jax.ShapeDtypeStruct((Bv, EMB, Tv), jnp.float32),
        scratch_types=[
            pltpu.VMEM((VOCAB,), jnp.float32),
            pltpu.VMEM((T,), jnp.int32),
            pltpu.VMEM((T,), jnp.int32),
            pltpu.VMEM((T,), jnp.float32),
            pltpu.VMEM((T,), jnp.float32),
            pltpu.VMEM((T,), jnp.float32),
            pltpu.SemaphoreType.DMA,
            pltpu.SemaphoreType.DMA,
            pltpu.SemaphoreType.DMA,
        ],
    )(tokT, x.astype(jnp.int32), posT)
    return outT.transpose(0, 2, 1)


# speedup vs baseline: 1.6691x; 1.0508x over previous
import functools

import jax
import jax.numpy as jnp
from jax import lax
from jax.experimental import pallas as pl
from jax.experimental.pallas import tpu as pltpu
from jax.experimental.pallas import tpu_sc as plsc

VOCAB = 100000
EMB = 64
B = 64
T = 2048

NUM_CORES = 2
NUM_SUBCORES = 16
NW = NUM_CORES * NUM_SUBCORES
FPW = EMB // NW
UNROLL = 4


def _embed_body(tokT_hbm, x_hbm, posT_hbm, out_hbm,
                row_v, idx0_v, idx1_v, pos_v, o0_v, o1_v,
                sem_r, sem_i, sem_s):
    wid = lax.axis_index("s") * NUM_CORES + lax.axis_index("c")
    idx = (idx0_v, idx1_v)
    o = (o0_v, o1_v)

    for f in range(FPW):
        e = wid * FPW + f
        pltpu.async_copy(tokT_hbm.at[e, pl.ds(0, VOCAB)], row_v, sem_r)
        pltpu.sync_copy(posT_hbm.at[e, pl.ds(0, T)], pos_v)
        pltpu.async_copy(x_hbm.at[0, pl.ds(0, T)], idx0_v, sem_i)
        pltpu.make_async_copy(
            tokT_hbm.at[e, pl.ds(0, VOCAB)], row_v, sem_r).wait()

        def batch_pair(g, _):
            for sub in range(2):
                b = 2 * g + sub
                cur = sub

                @pl.when(b < B - 1)
                def _prefetch():
                    pltpu.async_copy(
                        x_hbm.at[b + 1, pl.ds(0, T)], idx[1 - cur], sem_i)

                pltpu.make_async_copy(
                    x_hbm.at[b, pl.ds(0, T)], idx[cur], sem_i).wait()

                @pl.when(b >= 2)
                def _drain():
                    pltpu.make_async_copy(
                        o[cur], out_hbm.at[b - 2, e, pl.ds(0, T)],
                        sem_s).wait()

                iv = idx[cur]
                ov = o[cur]

                def body(i, _):
                    for u in range(UNROLL):
                        s = pl.ds(i * (16 * UNROLL) + u * 16, 16)
                        gth = plsc.load_gather(row_v, [iv[s]])
                        ov[s] = gth + pos_v[s]
                    return 0

                lax.fori_loop(0, T // (16 * UNROLL), body, 0)
                pltpu.async_copy(o[cur], out_hbm.at[b, e, pl.ds(0, T)], sem_s)
            return 0

        lax.fori_loop(0, B // 2, batch_pair, 0)
        pltpu.make_async_copy(
            o[0], out_hbm.at[B - 2, e, pl.ds(0, T)], sem_s).wait()
        pltpu.make_async_copy(
            o[1], out_hbm.at[B - 1, e, pl.ds(0, T)], sem_s).wait()


@jax.jit
def kernel(x, token_embedding_table, position_embedding_table):
    Bv, Tv = x.shape
    tokT = token_embedding_table.T
    posT = position_embedding_table[:Tv].T
    mesh = plsc.VectorSubcoreMesh(core_axis_name="c", subcore_axis_name="s")
    outT = pl.kernel(
        _embed_body,
        mesh=mesh,
        compiler_params=pltpu.CompilerParams(
            use_tc_tiling_on_sc=True, needs_layout_passes=False),
        out_type=jax.ShapeDtypeStruct((Bv, EMB, Tv), jnp.float32),
        scratch_types=[
            pltpu.VMEM((VOCAB,), jnp.float32),
            pltpu.VMEM((T,), jnp.int32),
            pltpu.VMEM((T,), jnp.int32),
            pltpu.VMEM((T,), jnp.float32),
            pltpu.VMEM((T,), jnp.float32),
            pltpu.VMEM((T,), jnp.float32),
            pltpu.SemaphoreType.DMA,
            pltpu.SemaphoreType.DMA,
            pltpu.SemaphoreType.DMA,
        ],
    )(tokT, x.astype(jnp.int32), posT)
    return outT.transpose(0, 2, 1)


# speedup vs baseline: 2.1155x; 1.2674x over previous
import functools

import jax
import jax.numpy as jnp
from jax import lax
from jax.experimental import pallas as pl
from jax.experimental.pallas import tpu as pltpu
from jax.experimental.pallas import tpu_sc as plsc

VOCAB = 100000
EMB = 64
B = 64
T = 2048

NUM_CORES = 2
NUM_SUBCORES = 16
NW = NUM_CORES * NUM_SUBCORES
FPW = EMB // NW
UNROLL = 4


def _embed_body(tokT_hbm, x_hbm, posT_hbm, out_hbm,
                row_v, idx0_v, idx1_v, pos_v, o0_v, o1_v,
                sem_r, sem_i, sem_s):
    wid = lax.axis_index("s") * NUM_CORES + lax.axis_index("c")
    idx = (idx0_v, idx1_v)
    o = (o0_v, o1_v)

    for f in range(FPW):
        e = wid * FPW + f
        pltpu.async_copy(tokT_hbm.at[e, pl.ds(0, VOCAB)], row_v, sem_r)
        pltpu.sync_copy(posT_hbm.at[e, pl.ds(0, T)], pos_v)
        pltpu.async_copy(x_hbm.at[0, pl.ds(0, T)], idx0_v, sem_i)
        pltpu.make_async_copy(
            tokT_hbm.at[e, pl.ds(0, VOCAB)], row_v, sem_r).wait()

        def batch_pair(g, _):
            for sub in range(2):
                b = 2 * g + sub
                cur = sub

                @pl.when(b < B - 1)
                def _prefetch():
                    pltpu.async_copy(
                        x_hbm.at[b + 1, pl.ds(0, T)], idx[1 - cur], sem_i)

                pltpu.make_async_copy(
                    x_hbm.at[b, pl.ds(0, T)], idx[cur], sem_i).wait()

                @pl.when(b >= 2)
                def _drain():
                    pltpu.make_async_copy(
                        o[cur], out_hbm.at[b - 2, e, pl.ds(0, T)],
                        sem_s).wait()

                iv = idx[cur]
                ov = o[cur]

                @plsc.parallel_loop(0, T // 16, unroll=UNROLL)
                def _gather_loop(i):
                    s = pl.ds(i * 16, 16)
                    gth = plsc.load_gather(row_v, [iv[s]])
                    ov[s] = gth + pos_v[s]
                pltpu.async_copy(o[cur], out_hbm.at[b, e, pl.ds(0, T)], sem_s)
            return 0

        lax.fori_loop(0, B // 2, batch_pair, 0)
        pltpu.make_async_copy(
            o[0], out_hbm.at[B - 2, e, pl.ds(0, T)], sem_s).wait()
        pltpu.make_async_copy(
            o[1], out_hbm.at[B - 1, e, pl.ds(0, T)], sem_s).wait()


@jax.jit
def kernel(x, token_embedding_table, position_embedding_table):
    Bv, Tv = x.shape
    tokT = token_embedding_table.T
    posT = position_embedding_table[:Tv].T
    mesh = plsc.VectorSubcoreMesh(core_axis_name="c", subcore_axis_name="s")
    outT = pl.kernel(
        _embed_body,
        mesh=mesh,
        compiler_params=pltpu.CompilerParams(
            use_tc_tiling_on_sc=True, needs_layout_passes=False),
        out_type=jax.ShapeDtypeStruct((Bv, EMB, Tv), jnp.float32),
        scratch_types=[
            pltpu.VMEM((VOCAB,), jnp.float32),
            pltpu.VMEM((T,), jnp.int32),
            pltpu.VMEM((T,), jnp.int32),
            pltpu.VMEM((T,), jnp.float32),
            pltpu.VMEM((T,), jnp.float32),
            pltpu.VMEM((T,), jnp.float32),
            pltpu.SemaphoreType.DMA,
            pltpu.SemaphoreType.DMA,
            pltpu.SemaphoreType.DMA,
        ],
    )(tokT, x.astype(jnp.int32), posT)
    return outT.transpose(0, 2, 1)


# speedup vs baseline: 2.6683x; 1.2613x over previous
import functools

import jax
import jax.numpy as jnp
from jax import lax
from jax.experimental import pallas as pl
from jax.experimental.pallas import tpu as pltpu
from jax.experimental.pallas import tpu_sc as plsc

VOCAB = 100000
EMB = 64
B = 64
T = 2048

NUM_CORES = 2
NUM_SUBCORES = 16
NW = NUM_CORES * NUM_SUBCORES
FPW = EMB // NW
UNROLL = 4
IG = 4
OG = 2
NQ = B // IG


def _embed_body(tokT_hbm, x_hbm, posT_hbm, out_hbm,
                row_v, idx0_v, idx1_v, pos_v, o0_v, o1_v,
                sem_r, sem_i, sem_s):
    wid = lax.axis_index("s") * NUM_CORES + lax.axis_index("c")
    idx = (idx0_v, idx1_v)
    o = (o0_v, o1_v)

    for f in range(FPW):
        e = wid * FPW + f
        pltpu.async_copy(tokT_hbm.at[e, pl.ds(0, VOCAB)], row_v, sem_r)
        pltpu.sync_copy(posT_hbm.at[e, pl.ds(0, T)], pos_v)
        pltpu.async_copy(x_hbm.at[pl.ds(0, IG), pl.ds(0, T)], idx0_v, sem_i)
        pltpu.make_async_copy(
            tokT_hbm.at[e, pl.ds(0, VOCAB)], row_v, sem_r).wait()

        def group_pair(gp, _):
            for par in range(2):
                q = 2 * gp + par
                b0 = q * IG
                iq = idx[par]

                @pl.when(q < NQ - 1)
                def _prefetch():
                    pltpu.async_copy(
                        x_hbm.at[pl.ds(b0 + IG, IG), pl.ds(0, T)],
                        idx[1 - par], sem_i)

                pltpu.make_async_copy(
                    x_hbm.at[pl.ds(b0, IG), pl.ds(0, T)], iq, sem_i).wait()

                for half in range(IG // OG):
                    bh = b0 + half * OG
                    ov = o[half]

                    @pl.when(q >= 1)
                    def _drain():
                        pltpu.make_async_copy(
                            ov,
                            out_hbm.at[pl.ds(bh - IG, OG), e, pl.ds(0, T)],
                            sem_s).wait()

                    for sub in range(OG):
                        s_b = half * OG + sub

                        @plsc.parallel_loop(0, T // 16, unroll=UNROLL)
                        def _gather_loop(i):
                            s = pl.ds(i * 16, 16)
                            gth = plsc.load_gather(row_v, [iq[s_b, s]])
                            ov[sub, s] = gth + pos_v[s]

                    pltpu.async_copy(
                        ov, out_hbm.at[pl.ds(bh, OG), e, pl.ds(0, T)],
                        sem_s)
            return 0

        lax.fori_loop(0, NQ // 2, group_pair, 0)
        pltpu.make_async_copy(
            o[0], out_hbm.at[pl.ds(B - IG, OG), e, pl.ds(0, T)],
            sem_s).wait()
        pltpu.make_async_copy(
            o[1], out_hbm.at[pl.ds(B - OG, OG), e, pl.ds(0, T)],
            sem_s).wait()


@jax.jit
def kernel(x, token_embedding_table, position_embedding_table):
    Bv, Tv = x.shape
    tokT = token_embedding_table.T
    posT = position_embedding_table[:Tv].T
    mesh = plsc.VectorSubcoreMesh(core_axis_name="c", subcore_axis_name="s")
    outT = pl.kernel(
        _embed_body,
        mesh=mesh,
        compiler_params=pltpu.CompilerParams(
            use_tc_tiling_on_sc=True, needs_layout_passes=False),
        out_type=jax.ShapeDtypeStruct((Bv, EMB, Tv), jnp.float32),
        scratch_types=[
            pltpu.VMEM((VOCAB,), jnp.float32),
            pltpu.VMEM((IG, T), jnp.int32),
            pltpu.VMEM((IG, T), jnp.int32),
            pltpu.VMEM((T,), jnp.float32),
            pltpu.VMEM((OG, T), jnp.float32),
            pltpu.VMEM((OG, T), jnp.float32),
            pltpu.SemaphoreType.DMA,
            pltpu.SemaphoreType.DMA,
            pltpu.SemaphoreType.DMA,
        ],
    )(tokT, x.astype(jnp.int32), posT)
    return outT.transpose(0, 2, 1)


# speedup vs baseline: 2.6794x; 1.0042x over previous
import functools

import jax
import jax.numpy as jnp
from jax import lax
from jax.experimental import pallas as pl
from jax.experimental.pallas import tpu as pltpu
from jax.experimental.pallas import tpu_sc as plsc

VOCAB = 100000
EMB = 64
B = 64
T = 2048

NUM_CORES = 2
NUM_SUBCORES = 16
NW = NUM_CORES * NUM_SUBCORES
FPW = EMB // NW
UNROLL = 8
IG = 4
OG = 2
NQ = B // IG


def _embed_body(tokT_hbm, x_hbm, posT_hbm, out_hbm,
                row_v, idx0_v, idx1_v, pos_v, o0_v, o1_v,
                sem_r, sem_i, sem_s):
    wid = lax.axis_index("s") * NUM_CORES + lax.axis_index("c")
    idx = (idx0_v, idx1_v)
    o = (o0_v, o1_v)

    for f in range(FPW):
        e = wid * FPW + f
        pltpu.async_copy(tokT_hbm.at[e, pl.ds(0, VOCAB)], row_v, sem_r)
        pltpu.sync_copy(posT_hbm.at[e, pl.ds(0, T)], pos_v)
        pltpu.async_copy(x_hbm.at[pl.ds(0, IG), pl.ds(0, T)], idx0_v, sem_i)
        pltpu.make_async_copy(
            tokT_hbm.at[e, pl.ds(0, VOCAB)], row_v, sem_r).wait()

        def group_pair(gp, _):
            for par in range(2):
                q = 2 * gp + par
                b0 = q * IG
                iq = idx[par]

                @pl.when(q < NQ - 1)
                def _prefetch():
                    pltpu.async_copy(
                        x_hbm.at[pl.ds(b0 + IG, IG), pl.ds(0, T)],
                        idx[1 - par], sem_i)

                pltpu.make_async_copy(
                    x_hbm.at[pl.ds(b0, IG), pl.ds(0, T)], iq, sem_i).wait()

                for half in range(IG // OG):
                    bh = b0 + half * OG
                    ov = o[half]

                    @pl.when(q >= 1)
                    def _drain():
                        pltpu.make_async_copy(
                            ov,
                            out_hbm.at[pl.ds(bh - IG, OG), e, pl.ds(0, T)],
                            sem_s).wait()

                    for sub in range(OG):
                        s_b = half * OG + sub

                        @plsc.parallel_loop(0, T // 16, unroll=UNROLL)
                        def _gather_loop(i):
                            s = pl.ds(i * 16, 16)
                            gth = plsc.load_gather(row_v, [iq[s_b, s]])
                            ov[sub, s] = gth + pos_v[s]

                    pltpu.async_copy(
                        ov, out_hbm.at[pl.ds(bh, OG), e, pl.ds(0, T)],
                        sem_s)
            return 0

        lax.fori_loop(0, NQ // 2, group_pair, 0)
        pltpu.make_async_copy(
            o[0], out_hbm.at[pl.ds(B - IG, OG), e, pl.ds(0, T)],
            sem_s).wait()
        pltpu.make_async_copy(
            o[1], out_hbm.at[pl.ds(B - OG, OG), e, pl.ds(0, T)],
            sem_s).wait()


@jax.jit
def kernel(x, token_embedding_table, position_embedding_table):
    Bv, Tv = x.shape
    tokT = token_embedding_table.T
    posT = position_embedding_table[:Tv].T
    mesh = plsc.VectorSubcoreMesh(core_axis_name="c", subcore_axis_name="s")
    outT = pl.kernel(
        _embed_body,
        mesh=mesh,
        compiler_params=pltpu.CompilerParams(
            use_tc_tiling_on_sc=True, needs_layout_passes=False),
        out_type=jax.ShapeDtypeStruct((Bv, EMB, Tv), jnp.float32),
        scratch_types=[
            pltpu.VMEM((VOCAB,), jnp.float32),
            pltpu.VMEM((IG, T), jnp.int32),
            pltpu.VMEM((IG, T), jnp.int32),
            pltpu.VMEM((T,), jnp.float32),
            pltpu.VMEM((OG, T), jnp.float32),
            pltpu.VMEM((OG, T), jnp.float32),
            pltpu.SemaphoreType.DMA,
            pltpu.SemaphoreType.DMA,
            pltpu.SemaphoreType.DMA,
        ],
    )(tokT, x.astype(jnp.int32), posT)
    return outT.transpose(0, 2, 1)
